# baseline (speedup 1.0000x reference)
import dataclasses
import functools

import jax
import jax.numpy as jnp
from jax import lax
from jax.experimental import pallas as pl
from jax.experimental.pallas import tpu as pltpu
from jax.experimental.pallas import tpu_sc as plsc

NN = 10000
EE = 320000
DW = 64
NC = 2
NS = 16
NW = NC * NS
RCH = 80
NRCH = NN // RCH
RRI = -(-NRCH // NS)
GCH = 128
TCH = EE // GCH
REM = TCH % NW
NPAIR = (TCH // NW + 2) // 2

_MESH = plsc.VectorSubcoreMesh(core_axis_name="c", subcore_axis_name="s")

_CP = pltpu.CompilerParams()
if "needs_layout_passes" in pltpu.CompilerParams.__dataclass_fields__:
    _CP = dataclasses.replace(_CP, needs_layout_passes=False)
if "use_tc_tiling_on_sc" in pltpu.CompilerParams.__dataclass_fields__:
    _CP = dataclasses.replace(_CP, use_tc_tiling_on_sc=False)


@functools.partial(
    pl.kernel,
    out_type=(
        jax.ShapeDtypeStruct((NC, NN, DW), jnp.float32),
        jax.ShapeDtypeStruct((NW, NN), jnp.float32),
    ),
    mesh=_MESH,
    scratch_types=[
        pltpu.VMEM_SHARED((NN, DW), jnp.float32),
        pltpu.VMEM((2, 2, GCH), jnp.int32),
        pltpu.VMEM((2, GCH, DW), jnp.float32),
        pltpu.VMEM((NN,), jnp.float32),
        pltpu.SemaphoreType.DMA,
        pltpu.SemaphoreType.DMA,
        pltpu.SemaphoreType.DMA,
        pltpu.SemaphoreType.DMA,
    ],
    compiler_params=_CP,
)
def _sc_segsum(x_hbm, ei_hbm, sum_hbm, cnt_hbm,
               acc_sh, idxb, msgs, cnt_loc,
               semg0, semg1, sems0, sems1):
    c = lax.axis_index("c")
    s = lax.axis_index("s")
    wid = s * NC + c
    nj = TCH // NW + jnp.where(wid < REM, 1, 0)

    @pl.loop(0, RCH)
    def _(r):
        @pl.loop(0, DW // 16)
        def _(j):
            msgs.at[0, r, pl.ds(j * 16, 16)][...] = jnp.zeros(
                (16,), jnp.float32)
    @pl.loop(0, NN // 16)
    def _(i):
        cnt_loc.at[pl.ds(i * 16, 16)][...] = jnp.zeros((16,), jnp.float32)

    @pl.loop(0, RRI)
    def _(i):
        k = s + i * NS
        @pl.when(k < NRCH)
        def _():
            pltpu.sync_copy(msgs.at[0, pl.ds(0, RCH)],
                            acc_sh.at[pl.ds(k * RCH, RCH)])
    plsc.subcore_barrier()

    def scat_wait(b, sems):
        pltpu.make_async_copy(msgs.at[b], acc_sh.at[idxb.at[b, 1]],
                              sems).wait()

    def fetch(j, b, semg, sems):
        @pl.when(j >= 2)
        def _():
            scat_wait(b, sems)
        pltpu.sync_copy(ei_hbm.at[wid + j * NW], idxb.at[b])
        pltpu.make_async_copy(x_hbm.at[idxb.at[b, 0]], msgs.at[b],
                              semg).start()
        @pl.loop(0, GCH // 16)
        def _(e):
            idxv = idxb[b, 1, pl.ds(e * 16, 16)]
            plsc.addupdate_scatter(cnt_loc, [idxv],
                                   jnp.ones((16,), jnp.float32))

    def drain(b, semg, sems):
        pltpu.make_async_copy(x_hbm.at[idxb.at[b, 0]], msgs.at[b],
                              semg).wait()
        pltpu.async_copy(msgs.at[b], acc_sh.at[idxb.at[b, 1]], sems,
                         add=True)

    fetch(0, 0, semg0, sems0)

    @pl.loop(0, NPAIR)
    def _(i):
        j0 = 2 * i
        @pl.when(j0 + 1 < nj)
        def _():
            fetch(j0 + 1, 1, semg1, sems1)
        @pl.when(j0 < nj)
        def _():
            drain(0, semg0, sems0)
        @pl.when(j0 + 2 < nj)
        def _():
            fetch(j0 + 2, 0, semg0, sems0)
        @pl.when(j0 + 1 < nj)
        def _():
            drain(1, semg1, sems1)

    scat_wait(0, sems0)
    scat_wait(1, sems1)
    pltpu.sync_copy(cnt_loc, cnt_hbm.at[wid])
    plsc.subcore_barrier()

    @pl.loop(0, RRI)
    def _(i):
        k = s + i * NS
        @pl.when(k < NRCH)
        def _():
            r0 = k * RCH
            pltpu.sync_copy(acc_sh.at[pl.ds(r0, RCH)],
                            sum_hbm.at[c, pl.ds(r0, RCH)])


BLK = 1000


def _dot(a, b):
    return jax.lax.dot(a, b, precision=jax.lax.Precision.HIGHEST,
                       preferred_element_type=jnp.float32)


def _lin_body(x_ref, w_ref, b_ref, o_ref):
    o_ref[...] = _dot(x_ref[...], w_ref[...]) + b_ref[...]


def _lin(x, w, b):
    n, d = x.shape
    o = w.shape[1]
    return pl.pallas_call(
        _lin_body,
        grid=(n // BLK,),
        in_specs=[
            pl.BlockSpec((BLK, d), lambda i: (i, 0)),
            pl.BlockSpec((d, o), lambda i: (0, 0)),
            pl.BlockSpec((1, o), lambda i: (0, 0)),
        ],
        out_specs=pl.BlockSpec((BLK, o), lambda i: (i, 0)),
        out_shape=jax.ShapeDtypeStruct((n, o), jnp.float32),
    )(x, w, b)


def _tc1_body(sa_ref, sb_ref, c_ref, xr_ref, wla_ref, wlb_ref, w2l_ref,
              h_ref, p2_ref):
    cnt = jnp.maximum(c_ref[...], 1.0)
    mean_a = (sa_ref[0] + sa_ref[1]) / cnt
    mean_b = (sb_ref[0] + sb_ref[1]) / cnt
    h = jnp.maximum(
        _dot(mean_a, wla_ref[...]) + _dot(mean_b, wlb_ref[...])
        + xr_ref[...], 0.0)
    h_ref[...] = h
    p2_ref[...] = _dot(h, w2l_ref[...])


def _tc1(sa, sb, cnt, xr, wla, wlb, w2l):
    return pl.pallas_call(
        _tc1_body,
        grid=(NN // BLK,),
        in_specs=[
            pl.BlockSpec((NC, BLK, DW), lambda i: (0, i, 0)),
            pl.BlockSpec((NC, BLK, DW), lambda i: (0, i, 0)),
            pl.BlockSpec((BLK, 1), lambda i: (i, 0)),
            pl.BlockSpec((BLK, 128), lambda i: (i, 0)),
            pl.BlockSpec((DW, 128), lambda i: (0, 0)),
            pl.BlockSpec((DW, 128), lambda i: (0, 0)),
            pl.BlockSpec((128, 64), lambda i: (0, 0)),
        ],
        out_specs=[
            pl.BlockSpec((BLK, 128), lambda i: (i, 0)),
            pl.BlockSpec((BLK, 64), lambda i: (i, 0)),
        ],
        out_shape=[
            jax.ShapeDtypeStruct((NN, 128), jnp.float32),
            jax.ShapeDtypeStruct((NN, 64), jnp.float32),
        ],
    )(sa, sb, cnt, xr, wla, wlb, w2l)


def _tc2_body(sp_ref, c_ref, hr_ref, z_ref, lsm_ref):
    cnt = jnp.maximum(c_ref[...], 1.0)
    z = (sp_ref[0] + sp_ref[1]) / cnt + hr_ref[...]
    z_ref[...] = z
    e = z - jnp.max(z, axis=1, keepdims=True)
    lsm_ref[...] = e - jnp.log(jnp.sum(jnp.exp(e), axis=1, keepdims=True))


def _tc2(sp, cnt, hr):
    return pl.pallas_call(
        _tc2_body,
        grid=(NN // BLK,),
        in_specs=[
            pl.BlockSpec((NC, BLK, 64), lambda i: (0, i, 0)),
            pl.BlockSpec((BLK, 1), lambda i: (i, 0)),
            pl.BlockSpec((BLK, 64), lambda i: (i, 0)),
        ],
        out_specs=[
            pl.BlockSpec((BLK, 64), lambda i: (i, 0)),
            pl.BlockSpec((BLK, 64), lambda i: (i, 0)),
        ],
        out_shape=[
            jax.ShapeDtypeStruct((NN, 64), jnp.float32),
            jax.ShapeDtypeStruct((NN, 64), jnp.float32),
        ],
    )(sp, cnt, hr)


def kernel(x, edge_index, W1_l, W1_r, b1, W2_l, W2_r, b2):
    ei3 = edge_index.reshape(2, TCH, GCH).transpose(1, 0, 2)
    xa = x[:, :DW]
    xb = x[:, DW:]
    wla = W1_l[:DW]
    wlb = W1_l[DW:]

    xr = _lin(x, W1_r, b1.reshape(1, -1))
    sa, cnts = _sc_segsum(xa, ei3)
    sb, _ = _sc_segsum(xb, ei3)
    cnt = jnp.sum(cnts, axis=0)[:, None]
    h, p2 = _tc1(sa, sb, cnt, xr, wla, wlb, W2_l)
    hr = _lin(h, W2_r, b2.reshape(1, -1))
    sp, _ = _sc_segsum(p2, ei3)
    z, lsm = _tc2(sp, cnt, hr)
    return (z, lsm)


# speedup vs baseline: 1.3500x; 1.3500x over previous
import dataclasses
import functools

import jax
import jax.numpy as jnp
from jax import lax
from jax.experimental import pallas as pl
from jax.experimental.pallas import tpu as pltpu
from jax.experimental.pallas import tpu_sc as plsc

NN = 10000
EE = 320000
NC = 2
NS = 16
NW = NC * NS
EPW = EE // NW
CHUNK = 80
NCHUNK = EPW // CHUNK
RCH = 80
NRCH = NN // RCH
RRI = -(-NRCH // NS)
CW = 16

_MESH = plsc.VectorSubcoreMesh(core_axis_name="c", subcore_axis_name="s")

_CP = pltpu.CompilerParams()
if "needs_layout_passes" in pltpu.CompilerParams.__dataclass_fields__:
    _CP = dataclasses.replace(_CP, needs_layout_passes=False)


def _fill_const(buf, rows, cols, val):
    @pl.loop(0, rows)
    def _(r):
        @pl.loop(0, cols // 16)
        def _(j):
            buf.at[r, pl.ds(j * 16, 16)][...] = jnp.full(
                (16,), val, jnp.float32)


GCH = 128
TCH = EE // GCH
NJB = TCH // NW
REM = TCH % NW
NQUAD = (NJB + 4) // 4


@functools.partial(
    pl.kernel,
    out_type=(
        jax.ShapeDtypeStruct((NC, NN, 128), jnp.float32),
        jax.ShapeDtypeStruct((NW, NN), jnp.float32),
    ),
    mesh=_MESH,
    scratch_types=[
        pltpu.VMEM_SHARED((NN, 128), jnp.float32),
        pltpu.VMEM((2, 2, 2, GCH), jnp.int32),
        pltpu.VMEM((2, GCH, 128), jnp.float32),
        pltpu.VMEM((NN,), jnp.float32),
        pltpu.SemaphoreType.DMA,
        pltpu.SemaphoreType.DMA,
        pltpu.SemaphoreType.DMA,
        pltpu.SemaphoreType.DMA,
        pltpu.SemaphoreType.DMA,
    ],
    compiler_params=_CP,
)
def _sc_segsum(x_hbm, ei_hbm, sum_hbm, cnt_hbm,
               acc_sh, idxb, msgs, cnt_loc,
               semg0, semg1, sems0, sems1, semi):
    c = lax.axis_index("c")
    s = lax.axis_index("s")
    wid = s * NC + c
    nj = NJB + jnp.where(wid < REM, 1, 0)
    cstart = wid * NJB + jnp.minimum(wid, REM)

    @pl.loop(0, RCH)
    def _(r):
        @pl.loop(0, 128 // 16)
        def _(j):
            msgs.at[0, r, pl.ds(j * 16, 16)][...] = jnp.zeros(
                (16,), jnp.float32)
    @pl.loop(0, NN // 16)
    def _(i):
        cnt_loc.at[pl.ds(i * 16, 16)][...] = jnp.zeros((16,), jnp.float32)

    @pl.loop(0, RRI)
    def _(i):
        k = s + i * NS
        @pl.when(k < NRCH)
        def _():
            pltpu.sync_copy(msgs.at[0, pl.ds(0, RCH)],
                            acc_sh.at[pl.ds(k * RCH, RCH)])
    plsc.subcore_barrier()

    def scat_wait(b, sems):
        pltpu.make_async_copy(msgs.at[b], acc_sh.at[idxb.at[0, 0, 1]],
                              sems).wait()

    def blk_start(bi, slot):
        pltpu.make_async_copy(ei_hbm.at[pl.ds(cstart + 2 * bi, 2)],
                              idxb.at[slot], semi).start()

    def blk_wait(slot):
        pltpu.make_async_copy(ei_hbm.at[pl.ds(cstart, 2)],
                              idxb.at[slot], semi).wait()

    def fetch(j, b, u, t, semg, sems):
        @pl.when(j >= 2)
        def _():
            scat_wait(b, sems)
        pltpu.make_async_copy(x_hbm.at[idxb.at[u, t, 0]], msgs.at[b],
                              semg).start()
        @pl.loop(0, GCH // 16)
        def _(e):
            idxv = idxb[u, t, 1, pl.ds(e * 16, 16)]
            plsc.addupdate_scatter(cnt_loc, [idxv],
                                   jnp.ones((16,), jnp.float32))

    def drain(b, u, t, semg, sems):
        pltpu.make_async_copy(x_hbm.at[idxb.at[0, 0, 0]], msgs.at[b],
                              semg).wait()
        pltpu.async_copy(msgs.at[b], acc_sh.at[idxb.at[u, t, 1]], sems,
                        add=True)

    def step(i, cur):
        j0 = 2 * i
        nxt = cur ^ 1
        @pl.when(j0 + 1 < nj)
        def _():
            fetch(j0 + 1, 1, cur, 1, semg1, sems1)
        @pl.when(j0 + 2 < nj)
        def _():
            blk_start(i + 1, nxt)
        @pl.when(j0 < nj)
        def _():
            drain(0, cur, 0, semg0, sems0)
        @pl.when(j0 + 2 < nj)
        def _():
            blk_wait(nxt)
            fetch(j0 + 2, 0, nxt, 0, semg0, sems0)
        @pl.when(j0 + 1 < nj)
        def _():
            drain(1, cur, 1, semg1, sems1)

    pltpu.sync_copy(ei_hbm.at[pl.ds(cstart, 2)], idxb.at[0])
    fetch(0, 0, 0, 0, semg0, sems0)

    @pl.loop(0, NQUAD)
    def _(q):
        step(2 * q, 0)
        step(2 * q + 1, 1)

    scat_wait(0, sems0)
    scat_wait(1, sems1)
    pltpu.sync_copy(cnt_loc, cnt_hbm.at[wid])
    plsc.subcore_barrier()

    @pl.loop(0, RRI)
    def _(i):
        k = s + i * NS
        @pl.when(k < NRCH)
        def _():
            r0 = k * RCH
            pltpu.sync_copy(acc_sh.at[pl.ds(r0, RCH)],
                            sum_hbm.at[c, pl.ds(r0, RCH)])


BLK = 1000


def _dot(a, b):
    return jax.lax.dot(a, b, precision=jax.lax.Precision.HIGHEST,
                       preferred_element_type=jnp.float32)


def _tc1_body(s1_ref, c_ref, x_ref, w1l_ref, w1r_ref, b1_ref, h_ref):
    cnt = jnp.maximum(c_ref[...], 1.0)
    mean = (s1_ref[0] + s1_ref[1]) / cnt
    h_ref[...] = jnp.maximum(
        _dot(mean, w1l_ref[...]) + _dot(x_ref[...], w1r_ref[...])
        + b1_ref[...], 0.0)


def _tc1(sums1, cnts, x, w1l, w1r, b1):
    return pl.pallas_call(
        _tc1_body,
        grid=(NN // BLK,),
        in_specs=[
            pl.BlockSpec((NC, BLK, 128), lambda i: (0, i, 0)),
            pl.BlockSpec((BLK, 1), lambda i: (i, 0)),
            pl.BlockSpec((BLK, 128), lambda i: (i, 0)),
            pl.BlockSpec((128, 128), lambda i: (0, 0)),
            pl.BlockSpec((128, 128), lambda i: (0, 0)),
            pl.BlockSpec((1, 128), lambda i: (0, 0)),
        ],
        out_specs=pl.BlockSpec((BLK, 128), lambda i: (i, 0)),
        out_shape=jax.ShapeDtypeStruct((NN, 128), jnp.float32),
    )(sums1, cnts, x, w1l, w1r, b1)


def _tc2_body(s2_ref, c_ref, h_ref, w2l_ref, w2r_ref, b2_ref,
              z_ref, lsm_ref):
    cnt = jnp.maximum(c_ref[...], 1.0)
    mean = (s2_ref[0] + s2_ref[1]) / cnt
    z = _dot(mean, w2l_ref[...]) + _dot(h_ref[...], w2r_ref[...]) \
        + b2_ref[...]
    z_ref[...] = z
    e = z - jnp.max(z, axis=1, keepdims=True)
    lsm_ref[...] = e - jnp.log(jnp.sum(jnp.exp(e), axis=1, keepdims=True))


def _tc2(sums2, cnts, h, w2l, w2r, b2):
    return pl.pallas_call(
        _tc2_body,
        grid=(NN // BLK,),
        in_specs=[
            pl.BlockSpec((NC, BLK, 128), lambda i: (0, i, 0)),
            pl.BlockSpec((BLK, 1), lambda i: (i, 0)),
            pl.BlockSpec((BLK, 128), lambda i: (i, 0)),
            pl.BlockSpec((128, 64), lambda i: (0, 0)),
            pl.BlockSpec((128, 64), lambda i: (0, 0)),
            pl.BlockSpec((1, 64), lambda i: (0, 0)),
        ],
        out_specs=[
            pl.BlockSpec((BLK, 64), lambda i: (i, 0)),
            pl.BlockSpec((BLK, 64), lambda i: (i, 0)),
        ],
        out_shape=[
            jax.ShapeDtypeStruct((NN, 64), jnp.float32),
            jax.ShapeDtypeStruct((NN, 64), jnp.float32),
        ],
    )(sums2, cnts, h, w2l, w2r, b2)


def kernel(x, edge_index, W1_l, W1_r, b1, W2_l, W2_r, b2):
    ei3 = edge_index.reshape(2, TCH, GCH).transpose(1, 0, 2)
    sums1, cnts = _sc_segsum(x, ei3)
    cnt = jnp.sum(cnts, axis=0)[:, None]
    h = _tc1(sums1, cnt, x, W1_l, W1_r, b1.reshape(1, -1))
    sums2, _ = _sc_segsum(h, ei3)
    z, lsm = _tc2(sums2, cnt, h, W2_l, W2_r, b2.reshape(1, -1))
    return (z, lsm)


# speedup vs baseline: 1.3817x; 1.0235x over previous
import dataclasses
import functools

import jax
import jax.numpy as jnp
from jax import lax
from jax.experimental import pallas as pl
from jax.experimental.pallas import tpu as pltpu
from jax.experimental.pallas import tpu_sc as plsc

NN = 10000
EE = 320000
NC = 2
NS = 16
NW = NC * NS
EPW = EE // NW
CHUNK = 80
NCHUNK = EPW // CHUNK
RCH = 80
NRCH = NN // RCH
RRI = -(-NRCH // NS)
CW = 16

_MESH = plsc.VectorSubcoreMesh(core_axis_name="c", subcore_axis_name="s")

_CP = pltpu.CompilerParams()
if "needs_layout_passes" in pltpu.CompilerParams.__dataclass_fields__:
    _CP = dataclasses.replace(_CP, needs_layout_passes=False)


def _fill_const(buf, rows, cols, val):
    @pl.loop(0, rows)
    def _(r):
        @pl.loop(0, cols // 16)
        def _(j):
            buf.at[r, pl.ds(j * 16, 16)][...] = jnp.full(
                (16,), val, jnp.float32)


GCH = 128
TCH = EE // GCH
NJB = TCH // NW
REM = TCH % NW
NQUAD = (NJB + 4) // 4


@functools.partial(
    pl.kernel,
    out_type=(
        jax.ShapeDtypeStruct((NC, NN, 128), jnp.float32),
        jax.ShapeDtypeStruct((NW, NN), jnp.float32),
    ),
    mesh=_MESH,
    scratch_types=[
        pltpu.VMEM_SHARED((NN, 128), jnp.float32),
        pltpu.VMEM((2, 2, 2, GCH), jnp.int32),
        pltpu.VMEM((2, GCH, 128), jnp.float32),
        pltpu.VMEM((NN,), jnp.float32),
        pltpu.SemaphoreType.DMA,
        pltpu.SemaphoreType.DMA,
        pltpu.SemaphoreType.DMA,
        pltpu.SemaphoreType.DMA,
        pltpu.SemaphoreType.DMA,
    ],
    compiler_params=_CP,
)
def _sc_segsum(x_hbm, ei_hbm, sum_hbm, cnt_hbm,
               acc_sh, idxb, msgs, cnt_loc,
               semg0, semg1, sems0, sems1, semi):
    c = lax.axis_index("c")
    s = lax.axis_index("s")
    wid = s * NC + c
    nj = NJB + jnp.where(wid < REM, 1, 0)
    cstart = wid * NJB + jnp.minimum(wid, REM)

    @pl.loop(0, RCH)
    def _(r):
        @pl.loop(0, 128 // 16)
        def _(j):
            msgs.at[0, r, pl.ds(j * 16, 16)][...] = jnp.zeros(
                (16,), jnp.float32)
    @pl.loop(0, NN // 16)
    def _(i):
        cnt_loc.at[pl.ds(i * 16, 16)][...] = jnp.zeros((16,), jnp.float32)

    @pl.loop(0, RRI)
    def _(i):
        k = s + i * NS
        @pl.when(k < NRCH)
        def _():
            pltpu.sync_copy(msgs.at[0, pl.ds(0, RCH)],
                            acc_sh.at[pl.ds(k * RCH, RCH)])
    plsc.subcore_barrier()

    def scat_wait(b, sems):
        pltpu.make_async_copy(msgs.at[b], acc_sh.at[idxb.at[0, 0, 1]],
                              sems).wait()

    def blk_start(bi, slot):
        pltpu.make_async_copy(ei_hbm.at[pl.ds(cstart + 2 * bi, 2)],
                              idxb.at[slot], semi).start()

    def blk_wait(slot):
        pltpu.make_async_copy(ei_hbm.at[pl.ds(cstart, 2)],
                              idxb.at[slot], semi).wait()

    def fetch(j, b, u, t, semg, sems):
        @pl.when(j >= 2)
        def _():
            scat_wait(b, sems)
        pltpu.make_async_copy(x_hbm.at[idxb.at[u, t, 0]], msgs.at[b],
                              semg).start()
        @pl.loop(0, GCH // 16)
        def _(e):
            idxv = idxb[u, t, 1, pl.ds(e * 16, 16)]
            plsc.addupdate_scatter(cnt_loc, [idxv],
                                   jnp.ones((16,), jnp.float32))

    def drain(b, u, t, semg, sems):
        pltpu.make_async_copy(x_hbm.at[idxb.at[0, 0, 0]], msgs.at[b],
                              semg).wait()
        pltpu.async_copy(msgs.at[b], acc_sh.at[idxb.at[u, t, 1]], sems,
                        add=True)

    def step(i, cur):
        j0 = 2 * i
        nxt = cur ^ 1
        @pl.when(j0 + 1 < nj)
        def _():
            fetch(j0 + 1, 1, cur, 1, semg1, sems1)
        @pl.when(j0 + 2 < nj)
        def _():
            blk_start(i + 1, nxt)
        @pl.when(j0 < nj)
        def _():
            drain(0, cur, 0, semg0, sems0)
        @pl.when(j0 + 2 < nj)
        def _():
            blk_wait(nxt)
            fetch(j0 + 2, 0, nxt, 0, semg0, sems0)
        @pl.when(j0 + 1 < nj)
        def _():
            drain(1, cur, 1, semg1, sems1)

    pltpu.sync_copy(ei_hbm.at[pl.ds(cstart, 2)], idxb.at[0])
    fetch(0, 0, 0, 0, semg0, sems0)

    @pl.loop(0, NQUAD)
    def _(q):
        step(2 * q, 0)
        step(2 * q + 1, 1)

    scat_wait(0, sems0)
    scat_wait(1, sems1)
    pltpu.sync_copy(cnt_loc, cnt_hbm.at[wid])
    plsc.subcore_barrier()

    @pl.loop(0, RRI)
    def _(i):
        k = s + i * NS
        @pl.when(k < NRCH)
        def _():
            r0 = k * RCH
            pltpu.sync_copy(acc_sh.at[pl.ds(r0, RCH)],
                            sum_hbm.at[c, pl.ds(r0, RCH)])


BLK = 1000


def _dot(a, b):
    return jax.lax.dot(a, b, precision=jax.lax.Precision.HIGHEST,
                       preferred_element_type=jnp.float32)


def _lin_body(x_ref, w_ref, b_ref, o_ref):
    o_ref[...] = _dot(x_ref[...], w_ref[...]) + b_ref[...]


def _lin(x, w, b):
    n, d = x.shape
    o = w.shape[1]
    return pl.pallas_call(
        _lin_body,
        grid=(n // BLK,),
        in_specs=[
            pl.BlockSpec((BLK, d), lambda i: (i, 0)),
            pl.BlockSpec((d, o), lambda i: (0, 0)),
            pl.BlockSpec((1, o), lambda i: (0, 0)),
        ],
        out_specs=pl.BlockSpec((BLK, o), lambda i: (i, 0)),
        out_shape=jax.ShapeDtypeStruct((n, o), jnp.float32),
    )(x, w, b)


def _tc1_body(s1_ref, c_ref, xr_ref, w1l_ref, h_ref):
    cnt = jnp.maximum(c_ref[...], 1.0)
    mean = (s1_ref[0] + s1_ref[1]) / cnt
    h_ref[...] = jnp.maximum(
        _dot(mean, w1l_ref[...]) + xr_ref[...], 0.0)


def _tc1(sums1, cnts, xr, w1l):
    return pl.pallas_call(
        _tc1_body,
        grid=(NN // BLK,),
        in_specs=[
            pl.BlockSpec((NC, BLK, 128), lambda i: (0, i, 0)),
            pl.BlockSpec((BLK, 1), lambda i: (i, 0)),
            pl.BlockSpec((BLK, 128), lambda i: (i, 0)),
            pl.BlockSpec((128, 128), lambda i: (0, 0)),
        ],
        out_specs=pl.BlockSpec((BLK, 128), lambda i: (i, 0)),
        out_shape=jax.ShapeDtypeStruct((NN, 128), jnp.float32),
    )(sums1, cnts, xr, w1l)


def _tc2_body(s2_ref, c_ref, hr_ref, w2l_ref, z_ref, lsm_ref):
    cnt = jnp.maximum(c_ref[...], 1.0)
    mean = (s2_ref[0] + s2_ref[1]) / cnt
    z = _dot(mean, w2l_ref[...]) + hr_ref[...]
    z_ref[...] = z
    e = z - jnp.max(z, axis=1, keepdims=True)
    lsm_ref[...] = e - jnp.log(jnp.sum(jnp.exp(e), axis=1, keepdims=True))


def _tc2(sums2, cnts, hr, w2l):
    return pl.pallas_call(
        _tc2_body,
        grid=(NN // BLK,),
        in_specs=[
            pl.BlockSpec((NC, BLK, 128), lambda i: (0, i, 0)),
            pl.BlockSpec((BLK, 1), lambda i: (i, 0)),
            pl.BlockSpec((BLK, 64), lambda i: (i, 0)),
            pl.BlockSpec((128, 64), lambda i: (0, 0)),
        ],
        out_specs=[
            pl.BlockSpec((BLK, 64), lambda i: (i, 0)),
            pl.BlockSpec((BLK, 64), lambda i: (i, 0)),
        ],
        out_shape=[
            jax.ShapeDtypeStruct((NN, 64), jnp.float32),
            jax.ShapeDtypeStruct((NN, 64), jnp.float32),
        ],
    )(sums2, cnts, hr, w2l)


def kernel(x, edge_index, W1_l, W1_r, b1, W2_l, W2_r, b2):
    ei3 = edge_index.reshape(2, TCH, GCH).transpose(1, 0, 2)
    xr = _lin(x, W1_r, b1.reshape(1, -1))
    sums1, cnts = _sc_segsum(x, ei3)
    cnt = jnp.sum(cnts, axis=0)[:, None]
    h = _tc1(sums1, cnt, xr, W1_l)
    hr = _lin(h, W2_r, b2.reshape(1, -1))
    sums2, _ = _sc_segsum(h, ei3)
    z, lsm = _tc2(sums2, cnt, hr, W2_l)
    return (z, lsm)
